# baseline (device time: 111654 ns/iter reference)
import jax
import jax.numpy as jnp
from jax import lax
from jax.experimental import pallas as pl
from jax.experimental.pallas import tpu as pltpu

N_DEV = 32
LOG2_N = 5
V_PER = 4096
N_IDX = 1024
D = 512
CHUNK = 1024


def kernel(table, idx):
    def body(table_ref, idx_ref, out_ref, recv_buf, send_sems, recv_sems):
        me = lax.axis_index("i")

        local = idx_ref[:] - me * V_PER
        local2d = local.reshape(N_IDX, 1)
        acc = jnp.zeros((N_IDX, D), jnp.float32)
        for j in range(V_PER // CHUNK):
            cols = lax.broadcasted_iota(jnp.int32, (N_IDX, CHUNK), 1) + j * CHUNK
            onehot = (cols == local2d).astype(jnp.bfloat16)
            t_chunk = table_ref[j * CHUNK:(j + 1) * CHUNK, :].astype(jnp.bfloat16)
            acc = acc + jnp.dot(onehot, t_chunk,
                                preferred_element_type=jnp.float32)
        out_ref[...] = acc.astype(jnp.bfloat16)

        for s in range(LOG2_N):
            partner = me ^ (1 << s)
            rdma = pltpu.make_async_remote_copy(
                src_ref=out_ref,
                dst_ref=recv_buf.at[s],
                send_sem=send_sems.at[s],
                recv_sem=recv_sems.at[s],
                device_id=(partner,),
                device_id_type=pl.DeviceIdType.MESH,
            )
            rdma.start()
            rdma.wait()
            out_ref[...] = out_ref[...] + recv_buf[s]

    return pl.pallas_call(
        body,
        out_shape=jax.ShapeDtypeStruct((N_IDX, D), jnp.bfloat16),
        in_specs=[
            pl.BlockSpec(memory_space=pltpu.VMEM),
            pl.BlockSpec(memory_space=pltpu.VMEM),
        ],
        out_specs=pl.BlockSpec(memory_space=pltpu.VMEM),
        scratch_shapes=[
            pltpu.VMEM((LOG2_N, N_IDX, D), jnp.bfloat16),
            pltpu.SemaphoreType.DMA((LOG2_N,)),
            pltpu.SemaphoreType.DMA((LOG2_N,)),
        ],
    )(table, idx)


# device time: 10034 ns/iter; 11.1276x vs baseline; 11.1276x over previous
import jax
import jax.numpy as jnp
from jax import lax
from jax.experimental import pallas as pl
from jax.experimental.pallas import tpu as pltpu

N_DEV = 32
BITS = (1, 2, 4, 8, 16)
V_PER = 4096
N_IDX = 1024
D = 512
CHUNK = 1024


def kernel(table, idx):
    def body(table_ref, idx_ref, out_ref, recv_buf, send_sems, recv_sems):
        me = lax.axis_index("i")

        barrier_sem = pltpu.get_barrier_semaphore()
        for b in BITS:
            pl.semaphore_signal(
                barrier_sem, inc=1,
                device_id=(me ^ b,), device_id_type=pl.DeviceIdType.MESH,
            )
        pl.semaphore_wait(barrier_sem, len(BITS))

        local = idx_ref[:] - me * V_PER
        local2d = local.reshape(N_IDX, 1)
        acc = jnp.zeros((N_IDX, D), jnp.float32)
        for j in range(V_PER // CHUNK):
            cols = lax.broadcasted_iota(jnp.int32, (N_IDX, CHUNK), 1) + j * CHUNK
            onehot = (cols == local2d).astype(jnp.bfloat16)
            t_chunk = table_ref[j * CHUNK:(j + 1) * CHUNK, :].astype(jnp.bfloat16)
            acc = acc + jnp.dot(onehot, t_chunk,
                                preferred_element_type=jnp.float32)
        out_ref[...] = acc.astype(jnp.bfloat16)

        lo = me * 0
        los = []
        for k, b in enumerate(BITS):
            half = (N_IDX // 2) >> k
            los.append(lo)
            partner = me ^ b
            keep_off = jnp.where((me & b) != 0, half, 0)
            give_lo = lo + (half - keep_off)
            rdma = pltpu.make_async_remote_copy(
                src_ref=out_ref.at[pl.ds(give_lo, half), :],
                dst_ref=recv_buf.at[k, pl.ds(0, half), :],
                send_sem=send_sems.at[k],
                recv_sem=recv_sems.at[k],
                device_id=(partner,),
                device_id_type=pl.DeviceIdType.MESH,
            )
            rdma.start()
            rdma.wait()
            keep_lo = lo + keep_off
            out_ref[pl.ds(keep_lo, half), :] = (
                out_ref[pl.ds(keep_lo, half), :] + recv_buf[k, :half, :]
            )
            lo = keep_lo

        for k in reversed(range(len(BITS))):
            size = (N_IDX // 2) >> k
            partner = me ^ BITS[k]
            j = len(BITS) + (len(BITS) - 1 - k)
            rdma = pltpu.make_async_remote_copy(
                src_ref=out_ref.at[pl.ds(lo, size), :],
                dst_ref=out_ref.at[pl.ds(lo, size), :],
                send_sem=send_sems.at[j],
                recv_sem=recv_sems.at[j],
                device_id=(partner,),
                device_id_type=pl.DeviceIdType.MESH,
            )
            rdma.start()
            rdma.wait()
            lo = los[k]

    return pl.pallas_call(
        body,
        out_shape=jax.ShapeDtypeStruct((N_IDX, D), jnp.bfloat16),
        in_specs=[
            pl.BlockSpec(memory_space=pltpu.VMEM),
            pl.BlockSpec(memory_space=pltpu.VMEM),
        ],
        out_specs=pl.BlockSpec(memory_space=pltpu.VMEM),
        scratch_shapes=[
            pltpu.VMEM((len(BITS), N_IDX // 2, D), jnp.bfloat16),
            pltpu.SemaphoreType.DMA((2 * len(BITS),)),
            pltpu.SemaphoreType.DMA((2 * len(BITS),)),
        ],
        compiler_params=pltpu.CompilerParams(collective_id=0),
    )(table, idx)
